# Initial kernel scaffold; baseline (speedup 1.0000x reference)
import jax
import jax.numpy as jnp
from jax.experimental import pallas as pl


def kernel(nodes, neigh_idx, neigh_w, feat_table):
    raise NotImplementedError("write your pallas kernel here")


# speedup vs baseline: 1.2323x; 1.2323x over previous
import functools

import jax
import jax.numpy as jnp
from jax import lax
from jax.experimental import pallas as pl
from jax.experimental.pallas import tpu as pltpu
from jax.experimental.pallas import tpu_sc as plsc

NC = 2
NS = 16
L = 16
NW = NC * NS


@functools.lru_cache(maxsize=None)
def _build(N, D, K, chunk, iw, aw):
    g = K + 1
    dc = D // L
    kc = K // L
    ng = chunk // L
    mesh = plsc.VectorSubcoreMesh(core_axis_name="c", subcore_axis_name="s")

    @functools.partial(
        pl.kernel,
        mesh=mesh,
        out_type=jax.ShapeDtypeStruct((NW * chunk * D,), jnp.float32),
        compiler_params=pltpu.CompilerParams(needs_layout_passes=False),
        scratch_types=[
            pltpu.VMEM((chunk * iw,), jnp.int32),
            pltpu.VMEM((chunk * K,), jnp.float32),
            pltpu.VMEM((chunk * aw,), jnp.float32),
            pltpu.VMEM((chunk * D,), jnp.float32),
            pltpu.VMEM((g, D), jnp.float32),
            pltpu.VMEM((g, D), jnp.float32),
            pltpu.SemaphoreType.DMA,
            pltpu.SemaphoreType.DMA,
        ],
    )
    def aggregate(idx_hbm, wt_hbm, table_hbm, out_hbm,
                  idx_v, wt_v, a_v, out_v, rows0, rows1, sem0, sem1):
        wid = lax.axis_index("s") * NC + lax.axis_index("c")
        base = wid * chunk
        pltpu.sync_copy(idx_hbm.at[pl.ds(base * iw, chunk * iw)], idx_v)
        pltpu.sync_copy(wt_hbm.at[pl.ds(base * K, chunk * K)], wt_v)

        lane_off = lax.iota(jnp.int32, L) * aw

        def norm_body(t, carry):
            wb = t * K * L
            wt = [wt_v[pl.ds(wb + j * L, L)] for j in range(K)]
            tot = wt[0]
            for j in range(1, K):
                tot = tot + wt[j]
            safe = jnp.where(tot == 0.0, jnp.float32(1.0), tot)
            inv_total = jnp.float32(1.0) / safe
            rs = tot * inv_total + jnp.float32(1.0)
            inv_rs = jnp.float32(1.0) / rs
            s = inv_total * inv_rs
            ab = t * L * aw
            for j in range(K):
                plsc.store_scatter(a_v, [lane_off + (ab + j)], wt[j] * s)
            plsc.store_scatter(a_v, [lane_off + (ab + K)], inv_rs)
            return carry

        lax.fori_loop(0, ng, norm_body, 0)

        def gather_start(i, buf, sem):
            pltpu.make_async_copy(
                table_hbm.at[idx_v.at[pl.ds(i * iw, g)]], buf, sem).start()

        def gather_wait(buf, sem):
            pltpu.make_async_copy(
                table_hbm.at[idx_v.at[pl.ds(0, g)]], buf, sem).wait()

        def compute(i, rows):
            ab = i * aw
            av = [a_v[pl.ds(ab + c * L, L)] for c in range(kc + 1)]
            als = [av[c][l] for c in range(kc) for l in range(L)]
            a_self = av[kc][0]
            acc = [rows[K, pl.ds(c * L, L)] * a_self for c in range(dc)]
            for j in range(K):
                for c in range(dc):
                    acc[c] = acc[c] + rows[j, pl.ds(c * L, L)] * als[j]
            for c in range(dc):
                out_v[pl.ds(i * D + c * L, L)] = acc[c]

        gather_start(0, rows0, sem0)
        gather_start(1, rows1, sem1)

        def body(t, carry):
            i = 2 * t
            gather_wait(rows0, sem0)
            compute(i, rows0)
            gather_start(jnp.minimum(i + 2, chunk - 1), rows0, sem0)
            gather_wait(rows1, sem1)
            compute(i + 1, rows1)
            gather_start(jnp.minimum(i + 3, chunk - 1), rows1, sem1)
            return carry

        lax.fori_loop(0, chunk // 2, body, 0)
        gather_wait(rows0, sem0)
        gather_wait(rows1, sem1)
        pltpu.sync_copy(out_v, out_hbm.at[pl.ds(base * D, chunk * D)])

    return aggregate


def kernel(nodes, neigh_idx, neigh_w, feat_table):
    B, K = neigh_idx.shape
    N, D = feat_table.shape
    iw = -(-(K + 1) // 8) * 8
    aw = -(-(K + 1) // L) * L
    chunk = -(-B // NW)
    chunk = -(-chunk // L) * L
    b_pad = chunk * NW
    idx = jnp.concatenate(
        [neigh_idx.astype(jnp.int32),
         nodes.astype(jnp.int32)[:, None],
         jnp.zeros((B, iw - K - 1), jnp.int32)], axis=1)
    idx = jnp.pad(idx, ((0, b_pad - B), (0, 0))).reshape(-1)
    w = jnp.pad(neigh_w.astype(jnp.float32), ((0, b_pad - B), (0, 0)))
    wt = w.reshape(b_pad // L, L, K).transpose(0, 2, 1).reshape(-1)
    out = _build(N, D, K, chunk, iw, aw)(idx, wt, feat_table)
    return out.reshape(b_pad, D)[:B]
